# baseline (speedup 1.0000x reference)
import functools

import jax
import jax.numpy as jnp
from jax import lax
from jax.experimental import pallas as pl
from jax.experimental.pallas import tpu as pltpu
from jax.experimental.pallas import tpu_sc as plsc

D = 64
EPS = 1e-8
THRESH = 0.2

_NC = 2
_NS = 16
_NW = _NC * _NS


_SUB = 80
_K = 5
_CH = _SUB * _K


@functools.lru_cache(maxsize=None)
def _make_sc_gather(E):
    epw = E // _NW
    nit = epw // _CH
    mesh = plsc.VectorSubcoreMesh(core_axis_name="c", subcore_axis_name="s")

    @functools.partial(
        pl.kernel,
        mesh=mesh,
        compiler_params=pltpu.CompilerParams(use_tc_tiling_on_sc=False),
        out_type=[
            jax.ShapeDtypeStruct((E, 2 * D), jnp.float32),
            jax.ShapeDtypeStruct((E,), jnp.int32),
            jax.ShapeDtypeStruct((E,), jnp.int32),
        ],
        scratch_types=[
            [pltpu.VMEM((_CH,), jnp.int32) for _ in range(2)],
            [pltpu.VMEM((_CH,), jnp.int32) for _ in range(2)],
            [pltpu.VMEM((_CH, D), jnp.float32) for _ in range(2)],
            [pltpu.VMEM((_CH, D), jnp.float32) for _ in range(2)],
            [pltpu.VMEM((_CH,), jnp.int32) for _ in range(2)],
            [pltpu.VMEM((_CH,), jnp.int32) for _ in range(2)],
            [pltpu.SemaphoreType.DMA for _ in range(2)],
            [pltpu.SemaphoreType.DMA for _ in range(2)],
            [pltpu.SemaphoreType.DMA for _ in range(2)],
        ],
    )
    def sc_gather(src_hbm, dst_hbm, feats_hbm, types_hbm,
                  fsfd_out, ts_out, td_out,
                  srcv, dstv, fsv, fdv, tsv, tdv, isem, gsem, wsem):
        wid = lax.axis_index("s") * _NC + lax.axis_index("c")
        erow0 = wid * epw

        def issue_idx(t, b):
            eb = erow0 + t * _CH
            pltpu.async_copy(src_hbm.at[pl.ds(eb, _CH)], srcv[b], isem[b])
            pltpu.async_copy(dst_hbm.at[pl.ds(eb, _CH)], dstv[b], isem[b])

        def wait_idx(b):
            pltpu.make_async_copy(src_hbm.at[pl.ds(0, _CH)], srcv[b], isem[b]).wait()
            pltpu.make_async_copy(dst_hbm.at[pl.ds(0, _CH)], dstv[b], isem[b]).wait()

        def issue_gathers(b):
            pltpu.async_copy(feats_hbm.at[srcv[b]], fsv[b], gsem[b])
            pltpu.async_copy(feats_hbm.at[dstv[b]], fdv[b], gsem[b])
            pltpu.async_copy(types_hbm.at[srcv[b]], tsv[b], gsem[b])
            pltpu.async_copy(types_hbm.at[dstv[b]], tdv[b], gsem[b])

        def wait_gathers(b):
            pltpu.make_async_copy(feats_hbm.at[srcv[b]], fsv[b], gsem[b]).wait()
            pltpu.make_async_copy(feats_hbm.at[dstv[b]], fdv[b], gsem[b]).wait()
            pltpu.make_async_copy(types_hbm.at[srcv[b]], tsv[b], gsem[b]).wait()
            pltpu.make_async_copy(types_hbm.at[dstv[b]], tdv[b], gsem[b]).wait()

        def issue_wb(t, b):
            eb = erow0 + t * _CH
            pltpu.async_copy(fsv[b], fsfd_out.at[pl.ds(eb, _CH), pl.ds(0, D)], wsem[b])
            pltpu.async_copy(fdv[b], fsfd_out.at[pl.ds(eb, _CH), pl.ds(D, D)], wsem[b])
            pltpu.async_copy(tsv[b], ts_out.at[pl.ds(eb, _CH)], wsem[b])
            pltpu.async_copy(tdv[b], td_out.at[pl.ds(eb, _CH)], wsem[b])

        def wait_wb(b):
            pltpu.make_async_copy(fsv[b], fsfd_out.at[pl.ds(0, _CH), pl.ds(0, D)], wsem[b]).wait()
            pltpu.make_async_copy(fdv[b], fsfd_out.at[pl.ds(0, _CH), pl.ds(D, D)], wsem[b]).wait()
            pltpu.make_async_copy(tsv[b], ts_out.at[pl.ds(0, _CH)], wsem[b]).wait()
            pltpu.make_async_copy(tdv[b], td_out.at[pl.ds(0, _CH)], wsem[b]).wait()

        def chunk_body(t, b):
            nb = 1 - b
            wait_gathers(b)
            issue_wb(t, b)

            @pl.when(t + 2 < nit)
            def _():
                issue_idx(t + 2, b)

            @pl.when(t + 1 < nit)
            def _():
                wait_idx(nb)

                @pl.when(t >= 1)
                def _():
                    wait_wb(nb)

                issue_gathers(nb)

        issue_idx(0, 0)
        wait_idx(0)
        issue_gathers(0)
        if nit > 1:
            issue_idx(1, 1)

        def loop_body(g, carry):
            chunk_body(2 * g, 0)
            chunk_body(2 * g + 1, 1)
            return carry

        if nit % 2:
            lax.fori_loop(0, (nit - 1) // 2, loop_body, 0)
            chunk_body(nit - 1, (nit - 1) % 2)
        else:
            lax.fori_loop(0, nit // 2, loop_body, 0)
        if nit >= 2:
            wait_wb((nit - 2) % 2)
        wait_wb((nit - 1) % 2)

    return sc_gather


def _tc_body(fsfd_ref, ts_ref, td_ref, q_ref, w1_ref, b1_ref,
             w2_ref, b2_ref, ebase_ref, ptv_ref, pti_ref,
             s_ref, sims_ref, tv_ref, ti_ref, bv_s, bi_s):
    i = pl.program_id(0)
    B = fsfd_ref.shape[0]
    TR = B // 128
    TC_ = 128

    fsfdT = fsfd_ref[...].T
    fsT = fsfdT[:D]
    fdT = fsfdT[D:]
    zT = jnp.concatenate([fsfdT, fsT * fdT], axis=0)
    hT = jnp.dot(w1_ref[...], zT, preferred_element_type=jnp.float32)
    hT = jnp.maximum(hT + b1_ref[...], 0.0)
    logit = jnp.dot(w2_ref[...], hT, preferred_element_type=jnp.float32)

    qc = q_ref[...]
    qn = qc / (jnp.sqrt(jnp.sum(qc * qc)) + EPS)
    fdq = jnp.dot(qn, fdT, preferred_element_type=jnp.float32)
    nrm2 = jnp.dot(jnp.ones((1, D), jnp.float32), fdT * fdT,
                   preferred_element_type=jnp.float32)

    logit2 = logit.reshape(TR, TC_)
    fdq2 = fdq.reshape(TR, TC_)
    nrm22 = nrm2.reshape(TR, TC_)

    s = jax.nn.sigmoid(logit2 + b2_ref[...])
    s_ref[0] = s
    sims = fdq2 / (jnp.sqrt(nrm22) + EPS)
    sims_ref[0] = sims

    mask = (s >= THRESH) & (ts_ref[0] == 0) & (td_ref[0] == 1)
    masked = jnp.where(mask, sims, -1e9)
    bmax = jnp.max(masked)
    idx2 = (lax.broadcasted_iota(jnp.int32, masked.shape, 0) * TC_
            + lax.broadcasted_iota(jnp.int32, masked.shape, 1))
    lidx = jnp.min(jnp.where(masked == bmax, idx2, jnp.int32(2**30)))

    @pl.when(i == 0)
    def _():
        bv_s[0] = ptv_ref[0, 0]
        bi_s[0] = pti_ref[0, 0]

    upd = bmax > bv_s[0]
    bv_s[0] = jnp.where(upd, bmax, bv_s[0])
    bi_s[0] = jnp.where(upd, ebase_ref[0, 0] + i * B + lidx, bi_s[0])
    tv_ref[0, 0] = bv_s[0]
    ti_ref[0, 0] = bi_s[0]


def _tc_call(E, B, interpret=False):
    nb = E // B
    RB = B // 128
    nrow = E // 128
    return pl.pallas_call(
        _tc_body,
        grid=(nb,),
        in_specs=[
            pl.BlockSpec((B, 2 * D), lambda i: (i, 0)),
            pl.BlockSpec((1, RB, 128), lambda i: (i, 0, 0)),
            pl.BlockSpec((1, RB, 128), lambda i: (i, 0, 0)),
            pl.BlockSpec((1, D), lambda i: (0, 0)),
            pl.BlockSpec((D, 3 * D), lambda i: (0, 0)),
            pl.BlockSpec((D, 1), lambda i: (0, 0)),
            pl.BlockSpec((1, D), lambda i: (0, 0)),
            pl.BlockSpec((1, 1), lambda i: (0, 0)),
            pl.BlockSpec((1, 1), lambda i: (0, 0), memory_space=pltpu.SMEM),
            pl.BlockSpec((1, 1), lambda i: (0, 0), memory_space=pltpu.SMEM),
            pl.BlockSpec((1, 1), lambda i: (0, 0), memory_space=pltpu.SMEM),
        ],
        out_specs=[
            pl.BlockSpec((1, RB, 128), lambda i: (i, 0, 0)),
            pl.BlockSpec((1, RB, 128), lambda i: (i, 0, 0)),
            pl.BlockSpec((1, 1), lambda i: (0, 0), memory_space=pltpu.SMEM),
            pl.BlockSpec((1, 1), lambda i: (0, 0), memory_space=pltpu.SMEM),
        ],
        out_shape=[
            jax.ShapeDtypeStruct((nb, RB, 128), jnp.float32),
            jax.ShapeDtypeStruct((nb, RB, 128), jnp.float32),
            jax.ShapeDtypeStruct((1, 1), jnp.float32),
            jax.ShapeDtypeStruct((1, 1), jnp.int32),
        ],
        scratch_shapes=[
            pltpu.SMEM((1,), jnp.float32),
            pltpu.SMEM((1,), jnp.int32),
        ],
        interpret=interpret,
    )


def kernel(query, node_feats, edge_index, node_types, W1, b1, W2, b2):
    E = edge_index.shape[1]
    src = edge_index[0].astype(jnp.int32)
    dst = edge_index[1].astype(jnp.int32)
    types32 = node_types.astype(jnp.int32)

    PIECES = [6, 6, 6, 6, 1]
    UNIT = E // sum(PIECES)
    B = 6400

    q2 = query.reshape(1, D)
    b1c = b1.reshape(D, 1)
    b2c = b2.reshape(1, 1)

    tv = jnp.full((1, 1), -3.4e38, jnp.float32)
    ti = jnp.zeros((1, 1), jnp.int32)
    s_parts, sims_parts = [], []
    e0 = 0
    for units in PIECES:
        Ep = units * UNIT
        nbp = Ep // B
        fsfd_g, ts_g, td_g = _make_sc_gather(Ep)(
            src[e0:e0 + Ep], dst[e0:e0 + Ep], node_feats, types32)
        ebase = jnp.full((1, 1), e0, jnp.int32)
        s3, sims3, tv, ti = _tc_call(Ep, B)(
            fsfd_g,
            ts_g.reshape(nbp, B // 128, 128), td_g.reshape(nbp, B // 128, 128),
            q2, W1, b1c, W2, b2c, ebase, tv, ti)
        s_parts.append(s3.reshape(Ep))
        sims_parts.append(sims3.reshape(Ep))
        e0 += Ep
    s_all = jnp.concatenate(s_parts)
    sims_all = jnp.concatenate(sims_parts)
    return s_all, sims_all, tv.reshape(1), ti.reshape(1)


# speedup vs baseline: 1.0095x; 1.0095x over previous
import functools

import jax
import jax.numpy as jnp
from jax import lax
from jax.experimental import pallas as pl
from jax.experimental.pallas import tpu as pltpu
from jax.experimental.pallas import tpu_sc as plsc

D = 64
EPS = 1e-8
THRESH = 0.2

_NC = 2
_NS = 16
_NW = _NC * _NS


_SUB = 80
_K = 5
_CH = _SUB * _K


@functools.lru_cache(maxsize=None)
def _make_sc_gather(E):
    epw = E // _NW
    nit = epw // _CH
    mesh = plsc.VectorSubcoreMesh(core_axis_name="c", subcore_axis_name="s")

    @functools.partial(
        pl.kernel,
        mesh=mesh,
        compiler_params=pltpu.CompilerParams(use_tc_tiling_on_sc=False),
        out_type=[
            jax.ShapeDtypeStruct((E, 2 * D), jnp.float32),
            jax.ShapeDtypeStruct((E,), jnp.int32),
            jax.ShapeDtypeStruct((E,), jnp.int32),
        ],
        scratch_types=[
            [pltpu.VMEM((_CH,), jnp.int32) for _ in range(2)],
            [pltpu.VMEM((_CH,), jnp.int32) for _ in range(2)],
            [pltpu.VMEM((_CH, D), jnp.float32) for _ in range(2)],
            [pltpu.VMEM((_CH, D), jnp.float32) for _ in range(2)],
            [pltpu.VMEM((_CH,), jnp.int32) for _ in range(2)],
            [pltpu.VMEM((_CH,), jnp.int32) for _ in range(2)],
            [pltpu.SemaphoreType.DMA for _ in range(2)],
            [pltpu.SemaphoreType.DMA for _ in range(2)],
            [pltpu.SemaphoreType.DMA for _ in range(2)],
        ],
    )
    def sc_gather(src_hbm, dst_hbm, feats_hbm, types_hbm, tok_hbm,
                  fsfd_out, ts_out, td_out,
                  srcv, dstv, fsv, fdv, tsv, tdv, isem, gsem, wsem):
        wid = lax.axis_index("s") * _NC + lax.axis_index("c")
        erow0 = wid * epw

        def issue_idx(t, b):
            eb = erow0 + t * _CH
            pltpu.async_copy(src_hbm.at[pl.ds(eb, _CH)], srcv[b], isem[b])
            pltpu.async_copy(dst_hbm.at[pl.ds(eb, _CH)], dstv[b], isem[b])

        def wait_idx(b):
            pltpu.make_async_copy(src_hbm.at[pl.ds(0, _CH)], srcv[b], isem[b]).wait()
            pltpu.make_async_copy(dst_hbm.at[pl.ds(0, _CH)], dstv[b], isem[b]).wait()

        def issue_gathers(b):
            pltpu.async_copy(feats_hbm.at[srcv[b]], fsv[b], gsem[b])
            pltpu.async_copy(feats_hbm.at[dstv[b]], fdv[b], gsem[b])
            pltpu.async_copy(types_hbm.at[srcv[b]], tsv[b], gsem[b])
            pltpu.async_copy(types_hbm.at[dstv[b]], tdv[b], gsem[b])

        def wait_gathers(b):
            pltpu.make_async_copy(feats_hbm.at[srcv[b]], fsv[b], gsem[b]).wait()
            pltpu.make_async_copy(feats_hbm.at[dstv[b]], fdv[b], gsem[b]).wait()
            pltpu.make_async_copy(types_hbm.at[srcv[b]], tsv[b], gsem[b]).wait()
            pltpu.make_async_copy(types_hbm.at[dstv[b]], tdv[b], gsem[b]).wait()

        def issue_wb(t, b):
            eb = erow0 + t * _CH
            pltpu.async_copy(fsv[b], fsfd_out.at[pl.ds(eb, _CH), pl.ds(0, D)], wsem[b])
            pltpu.async_copy(fdv[b], fsfd_out.at[pl.ds(eb, _CH), pl.ds(D, D)], wsem[b])
            pltpu.async_copy(tsv[b], ts_out.at[pl.ds(eb, _CH)], wsem[b])
            pltpu.async_copy(tdv[b], td_out.at[pl.ds(eb, _CH)], wsem[b])

        def wait_wb(b):
            pltpu.make_async_copy(fsv[b], fsfd_out.at[pl.ds(0, _CH), pl.ds(0, D)], wsem[b]).wait()
            pltpu.make_async_copy(fdv[b], fsfd_out.at[pl.ds(0, _CH), pl.ds(D, D)], wsem[b]).wait()
            pltpu.make_async_copy(tsv[b], ts_out.at[pl.ds(0, _CH)], wsem[b]).wait()
            pltpu.make_async_copy(tdv[b], td_out.at[pl.ds(0, _CH)], wsem[b]).wait()

        def chunk_body(t, b):
            nb = 1 - b
            wait_gathers(b)
            issue_wb(t, b)

            @pl.when(t + 2 < nit)
            def _():
                issue_idx(t + 2, b)

            @pl.when(t + 1 < nit)
            def _():
                wait_idx(nb)

                @pl.when(t >= 1)
                def _():
                    wait_wb(nb)

                issue_gathers(nb)

        issue_idx(0, 0)
        wait_idx(0)
        issue_gathers(0)
        if nit > 1:
            issue_idx(1, 1)

        def loop_body(g, carry):
            chunk_body(2 * g, 0)
            chunk_body(2 * g + 1, 1)
            return carry

        if nit % 2:
            lax.fori_loop(0, (nit - 1) // 2, loop_body, 0)
            chunk_body(nit - 1, (nit - 1) % 2)
        else:
            lax.fori_loop(0, nit // 2, loop_body, 0)
        if nit >= 2:
            wait_wb((nit - 2) % 2)
        wait_wb((nit - 1) % 2)

    return sc_gather


def _tc_body(fsfd_ref, ts_ref, td_ref, q_ref, w1_ref, b1_ref,
             w2_ref, b2_ref, ebase_ref, ptv_ref, pti_ref,
             s_ref, sims_ref, tv_ref, ti_ref, bv_s, bi_s):
    i = pl.program_id(0)
    B = fsfd_ref.shape[0]
    TR = B // 128
    TC_ = 128

    fsfdT = fsfd_ref[...].T
    fsT = fsfdT[:D]
    fdT = fsfdT[D:]
    zT = jnp.concatenate([fsfdT, fsT * fdT], axis=0)
    hT = jnp.dot(w1_ref[...], zT, preferred_element_type=jnp.float32)
    hT = jnp.maximum(hT + b1_ref[...], 0.0)
    logit = jnp.dot(w2_ref[...], hT, preferred_element_type=jnp.float32)

    qc = q_ref[...]
    qn = qc / (jnp.sqrt(jnp.sum(qc * qc)) + EPS)
    fdq = jnp.dot(qn, fdT, preferred_element_type=jnp.float32)
    nrm2 = jnp.dot(jnp.ones((1, D), jnp.float32), fdT * fdT,
                   preferred_element_type=jnp.float32)

    logit2 = logit.reshape(TR, TC_)
    fdq2 = fdq.reshape(TR, TC_)
    nrm22 = nrm2.reshape(TR, TC_)

    s = jax.nn.sigmoid(logit2 + b2_ref[...])
    s_ref[0] = s
    sims = fdq2 / (jnp.sqrt(nrm22) + EPS)
    sims_ref[0] = sims

    mask = (s >= THRESH) & (ts_ref[0] == 0) & (td_ref[0] == 1)
    masked = jnp.where(mask, sims, -1e9)
    bmax = jnp.max(masked)
    idx2 = (lax.broadcasted_iota(jnp.int32, masked.shape, 0) * TC_
            + lax.broadcasted_iota(jnp.int32, masked.shape, 1))
    lidx = jnp.min(jnp.where(masked == bmax, idx2, jnp.int32(2**30)))

    @pl.when(i == 0)
    def _():
        bv_s[0] = ptv_ref[0, 0]
        bi_s[0] = pti_ref[0, 0]

    upd = bmax > bv_s[0]
    bv_s[0] = jnp.where(upd, bmax, bv_s[0])
    bi_s[0] = jnp.where(upd, ebase_ref[0, 0] + i * B + lidx, bi_s[0])
    tv_ref[0, 0] = bv_s[0]
    ti_ref[0, 0] = bi_s[0]


def _tc_call(E, B, interpret=False):
    nb = E // B
    RB = B // 128
    nrow = E // 128
    return pl.pallas_call(
        _tc_body,
        grid=(nb,),
        in_specs=[
            pl.BlockSpec((B, 2 * D), lambda i: (i, 0)),
            pl.BlockSpec((1, RB, 128), lambda i: (i, 0, 0)),
            pl.BlockSpec((1, RB, 128), lambda i: (i, 0, 0)),
            pl.BlockSpec((1, D), lambda i: (0, 0)),
            pl.BlockSpec((D, 3 * D), lambda i: (0, 0)),
            pl.BlockSpec((D, 1), lambda i: (0, 0)),
            pl.BlockSpec((1, D), lambda i: (0, 0)),
            pl.BlockSpec((1, 1), lambda i: (0, 0)),
            pl.BlockSpec((1, 1), lambda i: (0, 0), memory_space=pltpu.SMEM),
            pl.BlockSpec((1, 1), lambda i: (0, 0), memory_space=pltpu.SMEM),
            pl.BlockSpec((1, 1), lambda i: (0, 0), memory_space=pltpu.SMEM),
        ],
        out_specs=[
            pl.BlockSpec((1, RB, 128), lambda i: (i, 0, 0)),
            pl.BlockSpec((1, RB, 128), lambda i: (i, 0, 0)),
            pl.BlockSpec((1, 1), lambda i: (0, 0), memory_space=pltpu.SMEM),
            pl.BlockSpec((1, 1), lambda i: (0, 0), memory_space=pltpu.SMEM),
        ],
        out_shape=[
            jax.ShapeDtypeStruct((nb, RB, 128), jnp.float32),
            jax.ShapeDtypeStruct((nb, RB, 128), jnp.float32),
            jax.ShapeDtypeStruct((1, 1), jnp.float32),
            jax.ShapeDtypeStruct((1, 1), jnp.int32),
        ],
        scratch_shapes=[
            pltpu.SMEM((1,), jnp.float32),
            pltpu.SMEM((1,), jnp.int32),
        ],
        interpret=interpret,
    )


def kernel(query, node_feats, edge_index, node_types, W1, b1, W2, b2):
    E = edge_index.shape[1]
    src = edge_index[0].astype(jnp.int32)
    dst = edge_index[1].astype(jnp.int32)
    types32 = node_types.astype(jnp.int32)

    PIECES = [6, 6, 6, 6, 1]
    UNIT = E // sum(PIECES)
    B = 6400

    q2 = query.reshape(1, D)
    b1c = b1.reshape(D, 1)
    b2c = b2.reshape(1, 1)

    tv = jnp.full((1, 1), -3.4e38, jnp.float32)
    ti = jnp.zeros((1, 1), jnp.int32)
    tv_hist = [tv]
    s_parts, sims_parts = [], []
    e0 = 0
    for units in PIECES:
        Ep = units * UNIT
        nbp = Ep // B
        fsfd_g, ts_g, td_g = _make_sc_gather(Ep)(
            src[e0:e0 + Ep], dst[e0:e0 + Ep], node_feats, types32,
            tv_hist[-2] if len(tv_hist) >= 2 else tv_hist[0])
        ebase = jnp.full((1, 1), e0, jnp.int32)
        s3, sims3, tv, ti = _tc_call(Ep, B)(
            fsfd_g,
            ts_g.reshape(nbp, B // 128, 128), td_g.reshape(nbp, B // 128, 128),
            q2, W1, b1c, W2, b2c, ebase, tv, ti)
        s_parts.append(s3.reshape(Ep))
        sims_parts.append(sims3.reshape(Ep))
        tv_hist.append(tv)
        e0 += Ep
    s_all = jnp.concatenate(s_parts)
    sims_all = jnp.concatenate(sims_parts)
    return s_all, sims_all, tv.reshape(1), ti.reshape(1)
